# baseline (speedup 1.0000x reference)
import functools

import jax
import jax.numpy as jnp
from jax import lax
from jax.experimental import pallas as pl
from jax.experimental.pallas import tpu as pltpu
from jax.experimental.pallas import tpu_sc as plsc

NC = 2
NS = 16
NW = NC * NS
CHUNK = 128
GROUP = 5
CORE0_SHARE = 1.0
DEGW = 128


def _acc_rows(N):
    return ((N + 1 + 127) // 128) * 128


def _make_deg_kernel(N, cpw0, cpw1):
    acc_rows = _acc_rows(N)
    zrows = acc_rows // NS
    mesh = plsc.VectorSubcoreMesh(core_axis_name="c", subcore_axis_name="s")

    @functools.partial(
        pl.kernel,
        out_type=jax.ShapeDtypeStruct((NC, acc_rows, DEGW), jnp.float32),
        mesh=mesh,
        scratch_types=[
            pltpu.VMEM((8, CHUNK), jnp.int32),
            pltpu.VMEM((CHUNK, DEGW), jnp.float32),
            pltpu.VMEM_SHARED((acc_rows, DEGW), jnp.float32),
            pltpu.SemaphoreType.DMA,
        ],
    )
    def deg_kernel(dst_hbm, ones_hbm, zeros_hbm, out_hbm, dst_v, ones_v,
                   acc_sh, sem):
        c = lax.axis_index("c")
        s = lax.axis_index("s")
        cpw_c = jnp.where(c == 0, cpw0, cpw1)
        base = jnp.where(c == 0, s * cpw0, NS * cpw0 + s * cpw1)
        pltpu.sync_copy(zeros_hbm.at[pl.ds(s * zrows, zrows)],
                        acc_sh.at[pl.ds(s * zrows, zrows)])
        pltpu.sync_copy(ones_hbm, ones_v)
        plsc.subcore_barrier()

        def body(o, carry):
            pltpu.sync_copy(dst_hbm.at[pl.ds(base + o * 8, 8)], dst_v)
            hs = [pltpu.async_copy(ones_v, acc_sh.at[dst_v.at[k]], sem,
                                   add=True) for k in range(8)]
            for h in hs:
                h.wait()
            return carry

        lax.fori_loop(0, cpw_c // 8, body, 0)
        plsc.subcore_barrier()
        pltpu.sync_copy(acc_sh.at[pl.ds(s * zrows, zrows)],
                        out_hbm.at[c, pl.ds(s * zrows, zrows)])

    return deg_kernel


def _make_scatter_kernel(N, D, cpw0, cpw1):
    acc_rows = _acc_rows(N)
    zrows = acc_rows // NS
    SG = 8
    GB = 2
    assert cpw0 % SG == 0 and cpw1 % SG == 0 and SG % GB == 0
    mesh = plsc.VectorSubcoreMesh(core_axis_name="c", subcore_axis_name="s")

    @functools.partial(
        pl.kernel,
        out_type=jax.ShapeDtypeStruct((NC, acc_rows, D), jnp.float32),
        mesh=mesh,
        scratch_types=[
            pltpu.VMEM((SG, CHUNK), jnp.int32),
            pltpu.VMEM((SG, CHUNK), jnp.int32),
            pltpu.VMEM((GB, CHUNK, D), jnp.float32),
            pltpu.VMEM_SHARED((acc_rows, D), jnp.float32),
            pltpu.SemaphoreType.DMA,
            pltpu.SemaphoreType.DMA,
            pltpu.SemaphoreType.DMA,
            pltpu.SemaphoreType.DMA,
        ],
    )
    def scat_kernel(g_hbm, src_hbm, dst_hbm, zeros_hbm, out_hbm,
                    src_v, dst_v, rows_v, acc_sh, g0, g1, s0, s1):
        c = lax.axis_index("c")
        s = lax.axis_index("s")
        cpw_c = jnp.where(c == 0, cpw0, cpw1)
        base = jnp.where(c == 0, s * cpw0, NS * cpw0 + s * cpw1)
        gsem = (g0, g1)
        ssem = (s0, s1)
        pltpu.sync_copy(zeros_hbm.at[pl.ds(s * zrows, zrows)],
                        acc_sh.at[pl.ds(s * zrows, zrows)])
        plsc.subcore_barrier()

        def gather(k):
            return pltpu.async_copy(
                g_hbm.at[src_v.at[k]], rows_v.at[k % GB], gsem[k % GB])

        def scatter(k):
            return pltpu.async_copy(
                rows_v.at[k % GB], acc_sh.at[dst_v.at[k]], ssem[k % GB],
                add=True)

        def body(o, carry):
            pltpu.sync_copy(src_hbm.at[pl.ds(base + o * SG, SG)], src_v)
            pltpu.sync_copy(dst_hbm.at[pl.ds(base + o * SG, SG)], dst_v)
            hg = [gather(0), gather(1)]
            hg[0].wait()
            hs = [scatter(0), None]
            for k in range(2, SG):
                b = k % GB
                hs[b].wait()
                hg[b] = gather(k)
                hg[1 - b].wait()
                hs[1 - b] = scatter(k - 1)
            hg[1].wait()
            hs[1] = scatter(SG - 1)
            hs[0].wait()
            hs[1].wait()
            return carry

        lax.fori_loop(0, cpw_c // SG, body, 0)
        plsc.subcore_barrier()
        pltpu.sync_copy(acc_sh.at[pl.ds(s * zrows, zrows)],
                        out_hbm.at[c, pl.ds(s * zrows, zrows)])

    return scat_kernel


def _enc_body(x_ref, w_ref, b_ref, h_ref):
    h_ref[...] = (jnp.dot(x_ref[...], w_ref[...],
                          preferred_element_type=jnp.float32) + b_ref[...])


def _g_body(h_ref, da_ref, db_ref, g_ref, dv_ref):
    h = h_ref[...]
    deg = (jnp.sum(da_ref[...], axis=1, keepdims=True)
           + jnp.sum(db_ref[...], axis=1, keepdims=True)) * (1.0 / DEGW) + 1.0
    dinv = lax.rsqrt(deg)
    dvb = jnp.broadcast_to(dinv, h.shape)
    dv_ref[...] = dvb
    g_ref[...] = dvb * h


def _layer_body(aa_ref, ab_ref, g_ref, h_ref, dv_ref, w_ref, b_ref,
                hn_ref, gn_ref):
    dv = dv_ref[...]
    agg = dv * (aa_ref[...] + ab_ref[...] + g_ref[...])
    z = jnp.maximum(
        jnp.dot(agg, w_ref[...], preferred_element_type=jnp.float32)
        + b_ref[...], 0.0) + h_ref[...]
    hn_ref[...] = z
    gn_ref[...] = dv * z


def _last_body(aa_ref, ab_ref, g_ref, h_ref, dv_ref, w_ref, b_ref,
               wd_ref, bd_ref, out_ref):
    dv = dv_ref[...]
    agg = dv * (aa_ref[...] + ab_ref[...] + g_ref[...])
    h3 = jnp.maximum(
        jnp.dot(agg, w_ref[...], preferred_element_type=jnp.float32)
        + b_ref[...], 0.0) + h_ref[...]
    out_ref[...] = (jnp.dot(h3, wd_ref[...], preferred_element_type=jnp.float32)
                    + bd_ref[...])


def _row_spec(br, d):
    return pl.BlockSpec((br, d), lambda i: (i, 0))


def _full_spec(shape):
    return pl.BlockSpec(shape, lambda i: tuple(0 for _ in shape))


def _tc_call(body, n_rows, br, in_arrays, in_specs, n_out, d):
    grid = (n_rows // br,)
    out_shape = [jax.ShapeDtypeStruct((n_rows, d), jnp.float32)] * n_out
    out_specs = [_row_spec(br, d)] * n_out
    if n_out == 1:
        out_shape, out_specs = out_shape[0], out_specs[0]
    return pl.pallas_call(
        body, grid=grid, in_specs=in_specs, out_specs=out_specs,
        out_shape=out_shape)(*in_arrays)


def kernel(x, edge_index, W_enc, b_enc, W0, b0, W1, b1, W2, b2, W_dec, b_dec):
    N, D = x.shape
    E = edge_index.shape[1]
    BR = 1000

    cpw = -(-E // (NW * CHUNK))
    cpw = ((cpw + 7) // 8) * 8
    pair = 2 * cpw
    cpw0 = max(0, min(pair, int(round(pair * CORE0_SHARE / 8)) * 8))
    cpw1 = pair - cpw0
    e_pad = NW * cpw * CHUNK
    src = edge_index[0].astype(jnp.int32)
    dst = edge_index[1].astype(jnp.int32)
    padn = e_pad - E
    src_p = jnp.concatenate(
        [src, jnp.zeros((padn,), jnp.int32)]).reshape(NW * cpw, CHUNK)
    dst_p = jnp.concatenate(
        [dst, jnp.full((padn,), N, jnp.int32)]).reshape(NW * cpw, CHUNK)

    acc_rows = _acc_rows(N)
    zeros_d = jnp.zeros((acc_rows, D), jnp.float32)
    ones_w = jnp.ones((CHUNK, DEGW), jnp.float32)

    deg_kernel = _make_deg_kernel(N, cpw0, cpw1)
    scat_kernel = _make_scatter_kernel(N, D, cpw0, cpw1)

    deg_parts = deg_kernel(dst_p, ones_w, zeros_d)[:, :N]

    b_enc2, b0_2, b1_2, b2_2, bd_2 = (
        v.reshape(1, D) for v in (b_enc, b0, b1, b2, b_dec))

    h = _tc_call(_enc_body, N, BR, (x, W_enc, b_enc2),
                 [_row_spec(BR, D), _full_spec((D, D)), _full_spec((1, D))],
                 1, D)

    g, dinvb = _tc_call(
        _g_body, N, BR, (h, deg_parts[0], deg_parts[1]),
        [_row_spec(BR, D), _row_spec(BR, DEGW), _row_spec(BR, DEGW)], 2, D)

    layer_specs = [_row_spec(BR, D)] * 5 + [_full_spec((D, D)), _full_spec((1, D))]
    for W, b2d in ((W0, b0_2), (W1, b1_2)):
        parts = scat_kernel(g, src_p, dst_p, zeros_d)[:, :N]
        h, g = _tc_call(_layer_body, N, BR,
                        (parts[0], parts[1], g, h, dinvb, W, b2d),
                        layer_specs, 2, D)

    parts = scat_kernel(g, src_p, dst_p, zeros_d)[:, :N]
    out = _tc_call(
        _last_body, N, BR,
        (parts[0], parts[1], g, h, dinvb, W2, b2_2, W_dec, bd_2),
        layer_specs + [_full_spec((D, D)), _full_spec((1, D))], 1, D)
    return out


# speedup vs baseline: 1.2812x; 1.2812x over previous
import functools

import jax
import jax.numpy as jnp
from jax import lax
from jax.experimental import pallas as pl
from jax.experimental.pallas import tpu as pltpu
from jax.experimental.pallas import tpu_sc as plsc

NC = 2
NS = 16
NW = NC * NS
CHUNK = 128
GROUP = 5
CORE0_SHARE = 0.8125
DEGW = 128


def _acc_rows(N):
    return ((N + 1 + 127) // 128) * 128


def _make_deg_kernel(N, cpw0, cpw1):
    acc_rows = _acc_rows(N)
    zrows = acc_rows // NS
    mesh = plsc.VectorSubcoreMesh(core_axis_name="c", subcore_axis_name="s")

    @functools.partial(
        pl.kernel,
        out_type=jax.ShapeDtypeStruct((NC, acc_rows, DEGW), jnp.float32),
        mesh=mesh,
        scratch_types=[
            pltpu.VMEM((8, CHUNK), jnp.int32),
            pltpu.VMEM((CHUNK, DEGW), jnp.float32),
            pltpu.VMEM_SHARED((acc_rows, DEGW), jnp.float32),
            pltpu.SemaphoreType.DMA,
        ],
    )
    def deg_kernel(dst_hbm, ones_hbm, zeros_hbm, out_hbm, dst_v, ones_v,
                   acc_sh, sem):
        c = lax.axis_index("c")
        s = lax.axis_index("s")
        cpw_c = jnp.where(c == 0, cpw0, cpw1)
        base = jnp.where(c == 0, s * cpw0, NS * cpw0 + s * cpw1)
        pltpu.sync_copy(zeros_hbm.at[pl.ds(s * zrows, zrows)],
                        acc_sh.at[pl.ds(s * zrows, zrows)])
        pltpu.sync_copy(ones_hbm, ones_v)
        plsc.subcore_barrier()

        def body(o, carry):
            pltpu.sync_copy(dst_hbm.at[pl.ds(base + o * 8, 8)], dst_v)
            hs = [pltpu.async_copy(ones_v, acc_sh.at[dst_v.at[k]], sem,
                                   add=True) for k in range(8)]
            for h in hs:
                h.wait()
            return carry

        lax.fori_loop(0, cpw_c // 8, body, 0)
        plsc.subcore_barrier()
        pltpu.sync_copy(acc_sh.at[pl.ds(s * zrows, zrows)],
                        out_hbm.at[c, pl.ds(s * zrows, zrows)])

    return deg_kernel


def _make_scatter_kernel(N, D, cpw0, cpw1):
    acc_rows = _acc_rows(N)
    zrows = acc_rows // NS
    SG = 8
    GB = 2
    assert cpw0 % SG == 0 and cpw1 % SG == 0 and SG % GB == 0
    mesh = plsc.VectorSubcoreMesh(core_axis_name="c", subcore_axis_name="s")

    @functools.partial(
        pl.kernel,
        out_type=jax.ShapeDtypeStruct((NC, acc_rows, D), jnp.float32),
        mesh=mesh,
        scratch_types=[
            pltpu.VMEM((SG, CHUNK), jnp.int32),
            pltpu.VMEM((SG, CHUNK), jnp.int32),
            pltpu.VMEM((GB, CHUNK, D), jnp.float32),
            pltpu.VMEM_SHARED((acc_rows, D), jnp.float32),
            pltpu.SemaphoreType.DMA,
            pltpu.SemaphoreType.DMA,
            pltpu.SemaphoreType.DMA,
            pltpu.SemaphoreType.DMA,
        ],
    )
    def scat_kernel(g_hbm, src_hbm, dst_hbm, zeros_hbm, out_hbm,
                    src_v, dst_v, rows_v, acc_sh, g0, g1, s0, s1):
        c = lax.axis_index("c")
        s = lax.axis_index("s")
        cpw_c = jnp.where(c == 0, cpw0, cpw1)
        base = jnp.where(c == 0, s * cpw0, NS * cpw0 + s * cpw1)
        gsem = (g0, g1)
        ssem = (s0, s1)
        pltpu.sync_copy(zeros_hbm.at[pl.ds(s * zrows, zrows)],
                        acc_sh.at[pl.ds(s * zrows, zrows)])
        plsc.subcore_barrier()

        def gather(k):
            return pltpu.async_copy(
                g_hbm.at[src_v.at[k]], rows_v.at[k % GB], gsem[k % GB])

        def scatter(k):
            return pltpu.async_copy(
                rows_v.at[k % GB], acc_sh.at[dst_v.at[k]], ssem[k % GB],
                add=True)

        def body(o, carry):
            pltpu.sync_copy(src_hbm.at[pl.ds(base + o * SG, SG)], src_v)
            pltpu.sync_copy(dst_hbm.at[pl.ds(base + o * SG, SG)], dst_v)
            hg = [gather(0), gather(1)]
            hg[0].wait()
            hs = [scatter(0), None]
            for k in range(2, SG):
                b = k % GB
                hs[b].wait()
                hg[b] = gather(k)
                hg[1 - b].wait()
                hs[1 - b] = scatter(k - 1)
            hg[1].wait()
            hs[1] = scatter(SG - 1)
            hs[0].wait()
            hs[1].wait()
            return carry

        lax.fori_loop(0, cpw_c // SG, body, 0)
        plsc.subcore_barrier()
        pltpu.sync_copy(acc_sh.at[pl.ds(s * zrows, zrows)],
                        out_hbm.at[c, pl.ds(s * zrows, zrows)])

    return scat_kernel


def _enc_body(x_ref, w_ref, b_ref, h_ref):
    h_ref[...] = (jnp.dot(x_ref[...], w_ref[...],
                          preferred_element_type=jnp.float32) + b_ref[...])


def _g_body(h_ref, da_ref, db_ref, g_ref, dv_ref):
    h = h_ref[...]
    deg = (jnp.sum(da_ref[...], axis=1, keepdims=True)
           + jnp.sum(db_ref[...], axis=1, keepdims=True)) * (1.0 / DEGW) + 1.0
    dinv = lax.rsqrt(deg)
    dvb = jnp.broadcast_to(dinv, h.shape)
    dv_ref[...] = dvb
    g_ref[...] = dvb * h


def _layer_body(aa_ref, ab_ref, g_ref, h_ref, dv_ref, w_ref, b_ref,
                hn_ref, gn_ref):
    dv = dv_ref[...]
    agg = dv * (aa_ref[...] + ab_ref[...] + g_ref[...])
    z = jnp.maximum(
        jnp.dot(agg, w_ref[...], preferred_element_type=jnp.float32)
        + b_ref[...], 0.0) + h_ref[...]
    hn_ref[...] = z
    gn_ref[...] = dv * z


def _last_body(aa_ref, ab_ref, g_ref, h_ref, dv_ref, w_ref, b_ref,
               wd_ref, bd_ref, out_ref):
    dv = dv_ref[...]
    agg = dv * (aa_ref[...] + ab_ref[...] + g_ref[...])
    h3 = jnp.maximum(
        jnp.dot(agg, w_ref[...], preferred_element_type=jnp.float32)
        + b_ref[...], 0.0) + h_ref[...]
    out_ref[...] = (jnp.dot(h3, wd_ref[...], preferred_element_type=jnp.float32)
                    + bd_ref[...])


def _row_spec(br, d):
    return pl.BlockSpec((br, d), lambda i: (i, 0))


def _full_spec(shape):
    return pl.BlockSpec(shape, lambda i: tuple(0 for _ in shape))


def _tc_call(body, n_rows, br, in_arrays, in_specs, n_out, d):
    grid = (n_rows // br,)
    out_shape = [jax.ShapeDtypeStruct((n_rows, d), jnp.float32)] * n_out
    out_specs = [_row_spec(br, d)] * n_out
    if n_out == 1:
        out_shape, out_specs = out_shape[0], out_specs[0]
    return pl.pallas_call(
        body, grid=grid, in_specs=in_specs, out_specs=out_specs,
        out_shape=out_shape)(*in_arrays)


def kernel(x, edge_index, W_enc, b_enc, W0, b0, W1, b1, W2, b2, W_dec, b_dec):
    N, D = x.shape
    E = edge_index.shape[1]
    BR = 1000

    cpw = -(-E // (NW * CHUNK))
    cpw = ((cpw + 7) // 8) * 8
    pair = 2 * cpw
    cpw0 = max(0, min(pair, int(round(pair * CORE0_SHARE / 8)) * 8))
    cpw1 = pair - cpw0
    e_pad = NW * cpw * CHUNK
    src = edge_index[0].astype(jnp.int32)
    dst = edge_index[1].astype(jnp.int32)
    padn = e_pad - E
    src_p = jnp.concatenate(
        [src, jnp.zeros((padn,), jnp.int32)]).reshape(NW * cpw, CHUNK)
    dst_p = jnp.concatenate(
        [dst, jnp.full((padn,), N, jnp.int32)]).reshape(NW * cpw, CHUNK)

    acc_rows = _acc_rows(N)
    zeros_d = jnp.zeros((acc_rows, D), jnp.float32)
    ones_w = jnp.ones((CHUNK, DEGW), jnp.float32)

    deg_kernel = _make_deg_kernel(N, cpw0, cpw1)
    scat_kernel = _make_scatter_kernel(N, D, cpw0, cpw1)

    deg_parts = deg_kernel(dst_p, ones_w, zeros_d)[:, :N]

    b_enc2, b0_2, b1_2, b2_2, bd_2 = (
        v.reshape(1, D) for v in (b_enc, b0, b1, b2, b_dec))

    h = _tc_call(_enc_body, N, BR, (x, W_enc, b_enc2),
                 [_row_spec(BR, D), _full_spec((D, D)), _full_spec((1, D))],
                 1, D)

    g, dinvb = _tc_call(
        _g_body, N, BR, (h, deg_parts[0], deg_parts[1]),
        [_row_spec(BR, D), _row_spec(BR, DEGW), _row_spec(BR, DEGW)], 2, D)

    layer_specs = [_row_spec(BR, D)] * 5 + [_full_spec((D, D)), _full_spec((1, D))]
    for W, b2d in ((W0, b0_2), (W1, b1_2)):
        parts = scat_kernel(g, src_p, dst_p, zeros_d)[:, :N]
        h, g = _tc_call(_layer_body, N, BR,
                        (parts[0], parts[1], g, h, dinvb, W, b2d),
                        layer_specs, 2, D)

    parts = scat_kernel(g, src_p, dst_p, zeros_d)[:, :N]
    out = _tc_call(
        _last_body, N, BR,
        (parts[0], parts[1], g, h, dinvb, W2, b2_2, W_dec, bd_2),
        layer_specs + [_full_spec((D, D)), _full_spec((1, D))], 1, D)
    return out
